# baseline (speedup 1.0000x reference)
import functools

import jax
import jax.numpy as jnp
from jax import lax
from jax.experimental import pallas as pl
from jax.experimental.pallas import tpu as pltpu
from jax.experimental.pallas import tpu_sc as plsc

N = 10000
E = 320000
D = 128
NUM_LAYERS = 6

NC = 2
NS = 16
NW = NC * NS
NPT = 320
NPAD = NW * NPT
CAP = 16384
CH = 8000
K = 128

_mesh = plsc.VectorSubcoreMesh(core_axis_name="c", subcore_axis_name="s")
_sc_params = pltpu.CompilerParams(needs_layout_passes=False)


def _worker_id():
    return lax.axis_index("c") * NS + lax.axis_index("s")


@functools.partial(
    pl.kernel,
    out_type=[
        jax.ShapeDtypeStruct((NW, CAP), jnp.int32),
        jax.ShapeDtypeStruct((NW, CAP), jnp.int32),
        jax.ShapeDtypeStruct((NW, 16), jnp.int32),
    ],
    mesh=_mesh,
    scratch_types=[
        pltpu.VMEM((CH,), jnp.int32),
        pltpu.VMEM((CH,), jnp.int32),
        pltpu.VMEM((CAP,), jnp.int32),
        pltpu.VMEM((CAP,), jnp.int32),
        pltpu.VMEM((16,), jnp.int32),
    ],
    compiler_params=_sc_params,
)
def _prep(esrc_hbm, edst_hbm, src_hbm, dl_hbm, cnt_hbm, s_v, d_v, fs_v, fd_v,
          c_v):
    wid = _worker_id()
    lo = wid * NPT
    hi = lo + NPT

    def zero_body(i, _):
        fs_v[pl.ds(i * 16, 16)] = jnp.zeros((16,), jnp.int32)
        fd_v[pl.ds(i * 16, 16)] = jnp.full((16,), NPT, jnp.int32)
        return 0

    lax.fori_loop(0, CAP // 16, zero_body, 0)

    def chunk_body(c, pos):
        pltpu.sync_copy(esrc_hbm.at[pl.ds(c * CH, CH)], s_v)
        pltpu.sync_copy(edst_hbm.at[pl.ds(c * CH, CH)], d_v)

        def vec_body(i, pos):
            dv = d_v[pl.ds(i * 16, 16)]
            sv = s_v[pl.ds(i * 16, 16)]
            m = (dv >= lo) & (dv < hi)
            key = jnp.where(m, jnp.int32(0), jnp.int32(1))
            dl = jnp.clip(dv - lo, 0, NPT - 1)
            code = (sv << 9) | dl
            _, scode = plsc.sort_key_val(key, code)
            fs_v[pl.ds(pos, 16)] = scode >> 9
            fd_v[pl.ds(pos, 16)] = scode & (512 - 1)
            pc = plsc.all_reduce_population_count(m)
            return jnp.minimum(pos + pc[0], CAP - 16)

        return lax.fori_loop(0, CH // 16, vec_body, pos)

    cnt = lax.fori_loop(0, E // CH, chunk_body, jnp.int32(0))

    fs_v[pl.ds(cnt, 16)] = jnp.zeros((16,), jnp.int32)
    fd_v[pl.ds(cnt, 16)] = jnp.full((16,), NPT, jnp.int32)

    c_v[...] = jnp.full((16,), cnt, jnp.int32)
    pltpu.sync_copy(c_v, cnt_hbm.at[wid])
    pltpu.sync_copy(fs_v, src_hbm.at[wid])
    pltpu.sync_copy(fd_v, dl_hbm.at[wid])


@functools.partial(
    pl.kernel,
    out_type=jax.ShapeDtypeStruct((NPAD, D), jnp.float32),
    mesh=_mesh,
    scratch_types=[
        pltpu.VMEM((CAP,), jnp.int32),
        pltpu.VMEM((CAP,), jnp.int32),
        pltpu.VMEM((16,), jnp.int32),
        [pltpu.VMEM(((NPT + 1) * 16,), jnp.float32) for _ in range(D // 16)],
        pltpu.VMEM((K, D), jnp.float32),
        pltpu.VMEM((K, D), jnp.float32),
        pltpu.SemaphoreType.DMA,
        pltpu.SemaphoreType.DMA,
    ],
    compiler_params=_sc_params,
)
def _segmax(h_hbm, src_hbm, dl_hbm, cnt_hbm, agg_hbm, src_v, dl_v, c_v,
            aggs_v, rows0_v, rows1_v, sem0, sem1):
    wid = _worker_id()

    pltpu.sync_copy(src_hbm.at[wid], src_v)
    pltpu.sync_copy(dl_hbm.at[wid], dl_v)
    pltpu.sync_copy(cnt_hbm.at[wid], c_v)
    cnt = c_v[pl.ds(0, 16)][0]

    ninf = jnp.full((16,), -jnp.inf, jnp.float32)

    def init_body(r, _):
        for j in range(D // 16):
            aggs_v[j][pl.ds(r * 16, 16)] = ninf
        return 0

    lax.fori_loop(0, NPT + 1, init_body, 0)

    nchunks = (cnt + K - 1) // K

    def start_gather(ci, buf, sem):
        pltpu.async_copy(h_hbm.at[src_v.at[pl.ds(ci * K, K)]], buf, sem)

    def wait_gather(buf, sem):
        pltpu.make_async_copy(h_hbm.at[src_v.at[pl.ds(0, K)]], buf,
                              sem).wait()

    @pl.when(nchunks > 0)
    def _():
        start_gather(0, rows0_v, sem0)

    @pl.when(nchunks > 1)
    def _():
        start_gather(1, rows1_v, sem1)

    def proc_chunk(ci, buf):
        ne = jnp.minimum(K, cnt - ci * K)
        ng = (ne + 15) // 16

        def gbody(g, _):
            dlv = dl_v[pl.ds(ci * K + g * 16, 16)]
            for lane in range(16):
                dl16 = dlv[lane] * 16
                e = g * 16 + lane
                for j in range(D // 16):
                    aggj = aggs_v[j]
                    aggj[pl.ds(dl16, 16)] = jnp.maximum(
                        aggj[pl.ds(dl16, 16)], buf[e, pl.ds(j * 16, 16)])
            return 0

        lax.fori_loop(0, ng, gbody, 0)

    def cbody(cc, _):
        c0 = cc * 2
        c1 = c0 + 1

        @pl.when(c0 < nchunks)
        def _():
            wait_gather(rows0_v, sem0)
            proc_chunk(c0, rows0_v)

            @pl.when(c0 + 2 < nchunks)
            def _():
                start_gather(c0 + 2, rows0_v, sem0)

        @pl.when(c1 < nchunks)
        def _():
            wait_gather(rows1_v, sem1)
            proc_chunk(c1, rows1_v)

            @pl.when(c1 + 2 < nchunks)
            def _():
                start_gather(c1 + 2, rows1_v, sem1)

        return 0

    lax.fori_loop(0, (nchunks + 1) // 2, cbody, 0)

    OCK = 64

    def out_chunk(oc, _):
        def row_body(r, _):
            for j in range(D // 16):
                v = aggs_v[j][pl.ds((oc * OCK + r) * 16, 16)]
                rows0_v[r, pl.ds(j * 16, 16)] = jnp.where(
                    v == -jnp.inf, 0.0, v)
            return 0

        lax.fori_loop(0, OCK, row_body, 0)
        pltpu.sync_copy(
            rows0_v.at[pl.ds(0, OCK)],
            agg_hbm.at[pl.ds(wid * NPT + oc * OCK, OCK)])
        return 0

    lax.fori_loop(0, NPT // OCK, out_chunk, 0)


BR = 1024


def _mm_body(relu, agg_ref, h_ref, wlt_ref, wrt_ref, bl_ref, o_ref):
    acc = jnp.dot(agg_ref[...], wlt_ref[...],
                  preferred_element_type=jnp.float32, precision="highest")
    acc += jnp.dot(h_ref[...], wrt_ref[...],
                   preferred_element_type=jnp.float32, precision="highest")
    acc += bl_ref[...]
    if relu:
        acc = jnp.maximum(acc, 0.0)
    o_ref[...] = acc


def _mm(agg, h, wlt, wrt, bl, relu):
    grid = (NPAD // BR,)
    return pl.pallas_call(
        functools.partial(_mm_body, relu),
        grid=grid,
        in_specs=[
            pl.BlockSpec((BR, D), lambda i: (i, 0)),
            pl.BlockSpec((BR, D), lambda i: (i, 0)),
            pl.BlockSpec((D, D), lambda i: (0, 0)),
            pl.BlockSpec((D, D), lambda i: (0, 0)),
            pl.BlockSpec((1, D), lambda i: (0, 0)),
        ],
        out_specs=pl.BlockSpec((BR, D), lambda i: (i, 0)),
        out_shape=jax.ShapeDtypeStruct((NPAD, D), jnp.float32),
    )(agg, h, wlt, wrt, bl)


def kernel(x, edge_index, params):
    h = jnp.pad(x, ((0, NPAD - N), (0, 0)))
    src_list, dl_list, counts = _prep(edge_index[0], edge_index[1])
    for i, (Wl, bl, Wr) in enumerate(params):
        agg = _segmax(h, src_list, dl_list, counts)
        h = _mm(agg, h, Wl.T, Wr.T, bl.reshape(1, D), i < NUM_LAYERS - 1)
    return h[:N]


# speedup vs baseline: 1.7184x; 1.7184x over previous
import functools

import jax
import jax.numpy as jnp
from jax import lax
from jax.experimental import pallas as pl
from jax.experimental.pallas import tpu as pltpu
from jax.experimental.pallas import tpu_sc as plsc

N = 10000
E = 320000
D = 128
NUM_LAYERS = 6

NC = 2
NS = 16
NW = NC * NS
NPT = 320
NPAD = NW * NPT
CAP = 16384
CH = 8000
K = 128

_mesh = plsc.VectorSubcoreMesh(core_axis_name="c", subcore_axis_name="s")
_sc_params = pltpu.CompilerParams(needs_layout_passes=False)


def _worker_id():
    return lax.axis_index("c") * NS + lax.axis_index("s")


@functools.partial(
    pl.kernel,
    out_type=[
        jax.ShapeDtypeStruct((NW, CAP), jnp.int32),
        jax.ShapeDtypeStruct((NW, CAP), jnp.int32),
        jax.ShapeDtypeStruct((NW, 16), jnp.int32),
    ],
    mesh=_mesh,
    scratch_types=[
        pltpu.VMEM((CH,), jnp.int32),
        pltpu.VMEM((CH,), jnp.int32),
        pltpu.VMEM((CAP,), jnp.int32),
        pltpu.VMEM((CAP,), jnp.int32),
        pltpu.VMEM((16,), jnp.int32),
    ],
    compiler_params=_sc_params,
)
def _prep(esrc_hbm, edst_hbm, src_hbm, dl_hbm, cnt_hbm, s_v, d_v, fs_v, fd_v,
          c_v):
    wid = _worker_id()
    lo = wid * NPT
    hi = lo + NPT

    def zero_body(i, _):
        fs_v[pl.ds(i * 16, 16)] = jnp.zeros((16,), jnp.int32)
        fd_v[pl.ds(i * 16, 16)] = jnp.full((16,), NPT, jnp.int32)
        return 0

    lax.fori_loop(0, CAP // 16, zero_body, 0)

    def chunk_body(c, pos):
        pltpu.sync_copy(esrc_hbm.at[pl.ds(c * CH, CH)], s_v)
        pltpu.sync_copy(edst_hbm.at[pl.ds(c * CH, CH)], d_v)

        def vec_body(i, pos):
            dv = d_v[pl.ds(i * 16, 16)]
            sv = s_v[pl.ds(i * 16, 16)]
            m = (dv >= lo) & (dv < hi)
            key = jnp.where(m, jnp.int32(0), jnp.int32(1))
            dl = jnp.clip(dv - lo, 0, NPT - 1)
            code = (sv << 9) | dl
            _, scode = plsc.sort_key_val(key, code)
            fs_v[pl.ds(pos, 16)] = scode >> 9
            fd_v[pl.ds(pos, 16)] = scode & (512 - 1)
            pc = plsc.all_reduce_population_count(m)
            return jnp.minimum(pos + pc[0], CAP - 16)

        return lax.fori_loop(0, CH // 16, vec_body, pos)

    cnt = lax.fori_loop(0, E // CH, chunk_body, jnp.int32(0))

    fs_v[pl.ds(cnt, 16)] = jnp.zeros((16,), jnp.int32)
    fd_v[pl.ds(cnt, 16)] = jnp.full((16,), NPT, jnp.int32)

    c_v[...] = jnp.full((16,), cnt, jnp.int32)
    pltpu.sync_copy(c_v, cnt_hbm.at[wid])
    pltpu.sync_copy(fs_v, src_hbm.at[wid])
    pltpu.sync_copy(fd_v, dl_hbm.at[wid])


@functools.partial(
    pl.kernel,
    out_type=jax.ShapeDtypeStruct((NPAD, D), jnp.float32),
    mesh=_mesh,
    scratch_types=[
        pltpu.VMEM((CAP,), jnp.int32),
        pltpu.VMEM((CAP,), jnp.int32),
        pltpu.VMEM((16,), jnp.int32),
        pltpu.VMEM(((NPT + 1) * D,), jnp.float32),
        pltpu.VMEM((K, D), jnp.float32),
        pltpu.VMEM((K, D), jnp.float32),
        pltpu.SemaphoreType.DMA,
        pltpu.SemaphoreType.DMA,
    ],
    compiler_params=_sc_params,
)
def _segmax(h_hbm, src_hbm, dl_hbm, cnt_hbm, agg_hbm, src_v, dl_v, c_v,
            agg_v, rows0_v, rows1_v, sem0, sem1):
    wid = _worker_id()

    pltpu.sync_copy(src_hbm.at[wid], src_v)
    pltpu.sync_copy(dl_hbm.at[wid], dl_v)
    pltpu.sync_copy(cnt_hbm.at[wid], c_v)
    cnt = c_v[pl.ds(0, 16)][0]

    ninf = jnp.full((16,), -jnp.inf, jnp.float32)

    def init_body(i, _):
        agg_v[pl.ds(i * 16, 16)] = ninf
        return 0

    lax.fori_loop(0, (NPT + 1) * D // 16, init_body, 0)

    nchunks = (cnt + K - 1) // K

    def start_gather(ci, buf, sem):
        pltpu.async_copy(h_hbm.at[src_v.at[pl.ds(ci * K, K)]], buf, sem)

    def wait_gather(buf, sem):
        pltpu.make_async_copy(h_hbm.at[src_v.at[pl.ds(0, K)]], buf,
                              sem).wait()

    @pl.when(nchunks > 0)
    def _():
        start_gather(0, rows0_v, sem0)

    @pl.when(nchunks > 1)
    def _():
        start_gather(1, rows1_v, sem1)

    def proc_chunk(ci, buf):
        ne = jnp.minimum(K, cnt - ci * K)
        ng = (ne + 15) // 16

        def gbody(g, _):
            dlv = dl_v[pl.ds(ci * K + g * 16, 16)] * D
            offs = [dlv[lane] for lane in range(16)]

            @plsc.parallel_loop(0, D // 16, unroll=D // 16)
            def _(j):
                for lane in range(16):
                    off = offs[lane] + j * 16
                    e = g * 16 + lane
                    agg_v[pl.ds(off, 16)] = jnp.maximum(
                        agg_v[pl.ds(off, 16)], buf[e, pl.ds(j * 16, 16)])

            return 0

        lax.fori_loop(0, ng, gbody, 0)

    def cbody(cc, _):
        c0 = cc * 2
        c1 = c0 + 1

        @pl.when(c0 < nchunks)
        def _():
            wait_gather(rows0_v, sem0)
            proc_chunk(c0, rows0_v)

            @pl.when(c0 + 2 < nchunks)
            def _():
                start_gather(c0 + 2, rows0_v, sem0)

        @pl.when(c1 < nchunks)
        def _():
            wait_gather(rows1_v, sem1)
            proc_chunk(c1, rows1_v)

            @pl.when(c1 + 2 < nchunks)
            def _():
                start_gather(c1 + 2, rows1_v, sem1)

        return 0

    lax.fori_loop(0, (nchunks + 1) // 2, cbody, 0)

    OCK = 64

    def out_chunk(oc, _):
        def row_body(r, _):
            for j in range(D // 16):
                v = agg_v[pl.ds((oc * OCK + r) * D + j * 16, 16)]
                rows0_v[r, pl.ds(j * 16, 16)] = jnp.where(
                    v == -jnp.inf, 0.0, v)
            return 0

        lax.fori_loop(0, OCK, row_body, 0)
        pltpu.sync_copy(
            rows0_v.at[pl.ds(0, OCK)],
            agg_hbm.at[pl.ds(wid * NPT + oc * OCK, OCK)])
        return 0

    lax.fori_loop(0, NPT // OCK, out_chunk, 0)


BR = 1024


def _mm_body(relu, agg_ref, h_ref, wlt_ref, wrt_ref, bl_ref, o_ref):
    acc = jnp.dot(agg_ref[...], wlt_ref[...],
                  preferred_element_type=jnp.float32, precision="highest")
    acc += jnp.dot(h_ref[...], wrt_ref[...],
                   preferred_element_type=jnp.float32, precision="highest")
    acc += bl_ref[...]
    if relu:
        acc = jnp.maximum(acc, 0.0)
    o_ref[...] = acc


def _mm(agg, h, wlt, wrt, bl, relu):
    grid = (NPAD // BR,)
    return pl.pallas_call(
        functools.partial(_mm_body, relu),
        grid=grid,
        in_specs=[
            pl.BlockSpec((BR, D), lambda i: (i, 0)),
            pl.BlockSpec((BR, D), lambda i: (i, 0)),
            pl.BlockSpec((D, D), lambda i: (0, 0)),
            pl.BlockSpec((D, D), lambda i: (0, 0)),
            pl.BlockSpec((1, D), lambda i: (0, 0)),
        ],
        out_specs=pl.BlockSpec((BR, D), lambda i: (i, 0)),
        out_shape=jax.ShapeDtypeStruct((NPAD, D), jnp.float32),
    )(agg, h, wlt, wrt, bl)


def kernel(x, edge_index, params):
    h = jnp.pad(x, ((0, NPAD - N), (0, 0)))
    src_list, dl_list, counts = _prep(edge_index[0], edge_index[1])
    for i, (Wl, bl, Wr) in enumerate(params):
        agg = _segmax(h, src_list, dl_list, counts)
        h = _mm(agg, h, Wl.T, Wr.T, bl.reshape(1, D), i < NUM_LAYERS - 1)
    return h[:N]


# speedup vs baseline: 1.9438x; 1.1312x over previous
import functools

import jax
import jax.numpy as jnp
from jax import lax
from jax.experimental import pallas as pl
from jax.experimental.pallas import tpu as pltpu
from jax.experimental.pallas import tpu_sc as plsc

N = 10000
E = 320000
D = 128
NUM_LAYERS = 6

NC = 2
NS = 16
NW = NC * NS
NPT = 320
NPAD = NW * NPT
CAP = 16384
CH = 8000
K = 128

_mesh = plsc.VectorSubcoreMesh(core_axis_name="c", subcore_axis_name="s")
_sc_params = pltpu.CompilerParams(needs_layout_passes=False)


def _worker_id():
    return lax.axis_index("c") * NS + lax.axis_index("s")


@functools.partial(
    pl.kernel,
    out_type=[
        jax.ShapeDtypeStruct((NW, CAP), jnp.int32),
        jax.ShapeDtypeStruct((NW, CAP), jnp.int32),
        jax.ShapeDtypeStruct((NW, 16), jnp.int32),
    ],
    mesh=_mesh,
    scratch_types=[
        pltpu.VMEM((CH,), jnp.int32),
        pltpu.VMEM((CH,), jnp.int32),
        pltpu.VMEM((CAP,), jnp.int32),
        pltpu.VMEM((CAP,), jnp.int32),
        pltpu.VMEM((16,), jnp.int32),
    ],
    compiler_params=_sc_params,
)
def _prep(esrc_hbm, edst_hbm, src_hbm, dl_hbm, cnt_hbm, s_v, d_v, fs_v, fd_v,
          c_v):
    wid = _worker_id()
    lo = wid * NPT
    hi = lo + NPT

    def zero_body(i, _):
        fs_v[pl.ds(i * 16, 16)] = jnp.zeros((16,), jnp.int32)
        fd_v[pl.ds(i * 16, 16)] = jnp.full((16,), NPT, jnp.int32)
        return 0

    lax.fori_loop(0, CAP // 16, zero_body, 0)

    UN = 4

    def chunk_body(c, pos):
        pltpu.sync_copy(esrc_hbm.at[pl.ds(c * CH, CH)], s_v)
        pltpu.sync_copy(edst_hbm.at[pl.ds(c * CH, CH)], d_v)

        def vec_body(i, pos):
            scodes, pcs = [], []
            for k in range(UN):
                dv = d_v[pl.ds((i * UN + k) * 16, 16)]
                sv = s_v[pl.ds((i * UN + k) * 16, 16)]
                m = (dv >= lo) & (dv < hi)
                key = jnp.where(m, jnp.int32(0), jnp.int32(1))
                dl = jnp.clip(dv - lo, 0, NPT - 1)
                code = (sv << 9) | dl
                _, scode = plsc.sort_key_val(key, code)
                scodes.append(scode)
                pcs.append(plsc.all_reduce_population_count(m))
            counts = [pc[0] for pc in pcs]
            for k in range(UN):
                fs_v[pl.ds(pos, 16)] = scodes[k] >> 9
                fd_v[pl.ds(pos, 16)] = scodes[k] & (512 - 1)
                pos = jnp.minimum(pos + counts[k], CAP - 16)
            return pos

        return lax.fori_loop(0, CH // (16 * UN), vec_body, pos)

    cnt = lax.fori_loop(0, E // CH, chunk_body, jnp.int32(0))

    fs_v[pl.ds(cnt, 16)] = jnp.zeros((16,), jnp.int32)
    fd_v[pl.ds(cnt, 16)] = jnp.full((16,), NPT, jnp.int32)

    c_v[...] = jnp.full((16,), cnt, jnp.int32)
    pltpu.sync_copy(c_v, cnt_hbm.at[wid])
    pltpu.sync_copy(fs_v, src_hbm.at[wid])
    pltpu.sync_copy(fd_v, dl_hbm.at[wid])


@functools.partial(
    pl.kernel,
    out_type=jax.ShapeDtypeStruct((NPAD, D), jnp.float32),
    mesh=_mesh,
    scratch_types=[
        pltpu.VMEM((CAP,), jnp.int32),
        pltpu.VMEM((CAP,), jnp.int32),
        pltpu.VMEM((16,), jnp.int32),
        pltpu.VMEM(((NPT + 1) * D,), jnp.float32),
        pltpu.VMEM((K, D), jnp.float32),
        pltpu.VMEM((K, D), jnp.float32),
        pltpu.SemaphoreType.DMA,
        pltpu.SemaphoreType.DMA,
    ],
    compiler_params=_sc_params,
)
def _segmax(h_hbm, src_hbm, dl_hbm, cnt_hbm, agg_hbm, src_v, dl_v, c_v,
            agg_v, rows0_v, rows1_v, sem0, sem1):
    wid = _worker_id()

    pltpu.sync_copy(src_hbm.at[wid], src_v)
    pltpu.sync_copy(dl_hbm.at[wid], dl_v)
    pltpu.sync_copy(cnt_hbm.at[wid], c_v)
    cnt = c_v[pl.ds(0, 16)][0]

    ninf = jnp.full((16,), -jnp.inf, jnp.float32)

    def init_body(i, _):
        agg_v[pl.ds(i * 16, 16)] = ninf
        return 0

    lax.fori_loop(0, (NPT + 1) * D // 16, init_body, 0)

    nchunks = (cnt + K - 1) // K

    def start_gather(ci, buf, sem):
        pltpu.async_copy(h_hbm.at[src_v.at[pl.ds(ci * K, K)]], buf, sem)

    def wait_gather(buf, sem):
        pltpu.make_async_copy(h_hbm.at[src_v.at[pl.ds(0, K)]], buf,
                              sem).wait()

    @pl.when(nchunks > 0)
    def _():
        start_gather(0, rows0_v, sem0)

    @pl.when(nchunks > 1)
    def _():
        start_gather(1, rows1_v, sem1)

    def proc_chunk(ci, buf):
        ne = jnp.minimum(K, cnt - ci * K)
        ng = (ne + 15) // 16

        def gbody(g, _):
            dlv = dl_v[pl.ds(ci * K + g * 16, 16)] * D
            offs = [dlv[lane] for lane in range(16)]

            @plsc.parallel_loop(0, D // 16, unroll=D // 16)
            def _(j):
                for lane in range(16):
                    off = offs[lane] + j * 16
                    e = g * 16 + lane
                    agg_v[pl.ds(off, 16)] = jnp.maximum(
                        agg_v[pl.ds(off, 16)], buf[e, pl.ds(j * 16, 16)])

            return 0

        lax.fori_loop(0, ng, gbody, 0)

    def cbody(cc, _):
        c0 = cc * 2
        c1 = c0 + 1

        @pl.when(c0 < nchunks)
        def _():
            wait_gather(rows0_v, sem0)
            proc_chunk(c0, rows0_v)

            @pl.when(c0 + 2 < nchunks)
            def _():
                start_gather(c0 + 2, rows0_v, sem0)

        @pl.when(c1 < nchunks)
        def _():
            wait_gather(rows1_v, sem1)
            proc_chunk(c1, rows1_v)

            @pl.when(c1 + 2 < nchunks)
            def _():
                start_gather(c1 + 2, rows1_v, sem1)

        return 0

    lax.fori_loop(0, (nchunks + 1) // 2, cbody, 0)

    OCK = 64

    def out_chunk(oc, _):
        def row_body(r, _):
            for j in range(D // 16):
                v = agg_v[pl.ds((oc * OCK + r) * D + j * 16, 16)]
                rows0_v[r, pl.ds(j * 16, 16)] = jnp.where(
                    v == -jnp.inf, 0.0, v)
            return 0

        lax.fori_loop(0, OCK, row_body, 0)
        pltpu.sync_copy(
            rows0_v.at[pl.ds(0, OCK)],
            agg_hbm.at[pl.ds(wid * NPT + oc * OCK, OCK)])
        return 0

    lax.fori_loop(0, NPT // OCK, out_chunk, 0)


BR = 1024


def _mm_body(relu, agg_ref, h_ref, wlt_ref, wrt_ref, bl_ref, o_ref):
    acc = jnp.dot(agg_ref[...], wlt_ref[...],
                  preferred_element_type=jnp.float32, precision="highest")
    acc += jnp.dot(h_ref[...], wrt_ref[...],
                   preferred_element_type=jnp.float32, precision="highest")
    acc += bl_ref[...]
    if relu:
        acc = jnp.maximum(acc, 0.0)
    o_ref[...] = acc


def _mm(agg, h, wlt, wrt, bl, relu):
    grid = (NPAD // BR,)
    return pl.pallas_call(
        functools.partial(_mm_body, relu),
        grid=grid,
        in_specs=[
            pl.BlockSpec((BR, D), lambda i: (i, 0)),
            pl.BlockSpec((BR, D), lambda i: (i, 0)),
            pl.BlockSpec((D, D), lambda i: (0, 0)),
            pl.BlockSpec((D, D), lambda i: (0, 0)),
            pl.BlockSpec((1, D), lambda i: (0, 0)),
        ],
        out_specs=pl.BlockSpec((BR, D), lambda i: (i, 0)),
        out_shape=jax.ShapeDtypeStruct((NPAD, D), jnp.float32),
    )(agg, h, wlt, wrt, bl)


def kernel(x, edge_index, params):
    h = jnp.pad(x, ((0, NPAD - N), (0, 0)))
    src_list, dl_list, counts = _prep(edge_index[0], edge_index[1])
    for i, (Wl, bl, Wr) in enumerate(params):
        agg = _segmax(h, src_list, dl_list, counts)
        h = _mm(agg, h, Wl.T, Wr.T, bl.reshape(1, D), i < NUM_LAYERS - 1)
    return h[:N]


# speedup vs baseline: 1.9891x; 1.0233x over previous
import functools

import jax
import jax.numpy as jnp
from jax import lax
from jax.experimental import pallas as pl
from jax.experimental.pallas import tpu as pltpu
from jax.experimental.pallas import tpu_sc as plsc

N = 10000
E = 320000
D = 128
NUM_LAYERS = 6

NC = 2
NS = 16
NW = NC * NS
NPT = 320
NPAD = NW * NPT
CAP = 16384
CH = 6400
K = 128

_mesh = plsc.VectorSubcoreMesh(core_axis_name="c", subcore_axis_name="s")
_sc_params = pltpu.CompilerParams(needs_layout_passes=False)


def _worker_id():
    return lax.axis_index("c") * NS + lax.axis_index("s")


@functools.partial(
    pl.kernel,
    out_type=[
        jax.ShapeDtypeStruct((NW, CAP), jnp.int32),
        jax.ShapeDtypeStruct((NW, CAP), jnp.int32),
        jax.ShapeDtypeStruct((NW, 16), jnp.int32),
    ],
    mesh=_mesh,
    scratch_types=[
        pltpu.VMEM((CH,), jnp.int32),
        pltpu.VMEM((CH,), jnp.int32),
        pltpu.VMEM((CAP,), jnp.int32),
        pltpu.VMEM((CAP,), jnp.int32),
        pltpu.VMEM((16,), jnp.int32),
    ],
    compiler_params=_sc_params,
)
def _prep(esrc_hbm, edst_hbm, src_hbm, dl_hbm, cnt_hbm, s_v, d_v, fs_v, fd_v,
          c_v):
    wid = _worker_id()
    lo = wid * NPT
    hi = lo + NPT

    def zero_body(i, _):
        fs_v[pl.ds(i * 16, 16)] = jnp.zeros((16,), jnp.int32)
        fd_v[pl.ds(i * 16, 16)] = jnp.full((16,), NPT, jnp.int32)
        return 0

    lax.fori_loop(0, CAP // 16, zero_body, 0)

    UN = 8

    def chunk_body(c, pos):
        pltpu.sync_copy(esrc_hbm.at[pl.ds(c * CH, CH)], s_v)
        pltpu.sync_copy(edst_hbm.at[pl.ds(c * CH, CH)], d_v)

        def vec_body(i, pos):
            scodes, pcs = [], []
            for k in range(UN):
                dv = d_v[pl.ds((i * UN + k) * 16, 16)]
                sv = s_v[pl.ds((i * UN + k) * 16, 16)]
                m = (dv >= lo) & (dv < hi)
                key = jnp.where(m, jnp.int32(0), jnp.int32(1))
                dl = jnp.clip(dv - lo, 0, NPT - 1)
                code = (sv << 9) | dl
                _, scode = plsc.sort_key_val(key, code)
                scodes.append(scode)
                pcs.append(plsc.all_reduce_population_count(m))
            counts = [pc[0] for pc in pcs]
            for k in range(UN):
                fs_v[pl.ds(pos, 16)] = scodes[k] >> 9
                fd_v[pl.ds(pos, 16)] = scodes[k] & (512 - 1)
                pos = jnp.minimum(pos + counts[k], CAP - 16)
            return pos

        return lax.fori_loop(0, CH // (16 * UN), vec_body, pos)

    cnt = lax.fori_loop(0, E // CH, chunk_body, jnp.int32(0))

    fs_v[pl.ds(cnt, 16)] = jnp.zeros((16,), jnp.int32)
    fd_v[pl.ds(cnt, 16)] = jnp.full((16,), NPT, jnp.int32)

    c_v[...] = jnp.full((16,), cnt, jnp.int32)
    pltpu.sync_copy(c_v, cnt_hbm.at[wid])
    pltpu.sync_copy(fs_v, src_hbm.at[wid])
    pltpu.sync_copy(fd_v, dl_hbm.at[wid])


@functools.partial(
    pl.kernel,
    out_type=jax.ShapeDtypeStruct((NPAD, D), jnp.float32),
    mesh=_mesh,
    scratch_types=[
        pltpu.VMEM((CAP,), jnp.int32),
        pltpu.VMEM((CAP,), jnp.int32),
        pltpu.VMEM((16,), jnp.int32),
        pltpu.VMEM(((NPT + 1) * D,), jnp.float32),
        pltpu.VMEM((K, D), jnp.float32),
        pltpu.VMEM((K, D), jnp.float32),
        pltpu.SemaphoreType.DMA,
        pltpu.SemaphoreType.DMA,
    ],
    compiler_params=_sc_params,
)
def _segmax(h_hbm, src_hbm, dl_hbm, cnt_hbm, agg_hbm, src_v, dl_v, c_v,
            agg_v, rows0_v, rows1_v, sem0, sem1):
    wid = _worker_id()

    pltpu.sync_copy(src_hbm.at[wid], src_v)
    pltpu.sync_copy(dl_hbm.at[wid], dl_v)
    pltpu.sync_copy(cnt_hbm.at[wid], c_v)
    cnt = c_v[pl.ds(0, 16)][0]

    ninf = jnp.full((16,), -jnp.inf, jnp.float32)

    def init_body(i, _):
        agg_v[pl.ds(i * 16, 16)] = ninf
        return 0

    lax.fori_loop(0, (NPT + 1) * D // 16, init_body, 0)

    nchunks = (cnt + K - 1) // K

    def start_gather(ci, buf, sem):
        pltpu.async_copy(h_hbm.at[src_v.at[pl.ds(ci * K, K)]], buf, sem)

    def wait_gather(buf, sem):
        pltpu.make_async_copy(h_hbm.at[src_v.at[pl.ds(0, K)]], buf,
                              sem).wait()

    @pl.when(nchunks > 0)
    def _():
        start_gather(0, rows0_v, sem0)

    @pl.when(nchunks > 1)
    def _():
        start_gather(1, rows1_v, sem1)

    def proc_chunk(ci, buf):
        ne = jnp.minimum(K, cnt - ci * K)
        ng = (ne + 15) // 16

        def gbody(g, _):
            dlv = dl_v[pl.ds(ci * K + g * 16, 16)] * D
            offs = [dlv[lane] for lane in range(16)]

            @plsc.parallel_loop(0, D // 16, unroll=D // 16)
            def _(j):
                for lane in range(16):
                    off = offs[lane] + j * 16
                    e = g * 16 + lane
                    agg_v[pl.ds(off, 16)] = jnp.maximum(
                        agg_v[pl.ds(off, 16)], buf[e, pl.ds(j * 16, 16)])

            return 0

        lax.fori_loop(0, ng, gbody, 0)

    def cbody(cc, _):
        c0 = cc * 2
        c1 = c0 + 1

        @pl.when(c0 < nchunks)
        def _():
            wait_gather(rows0_v, sem0)
            proc_chunk(c0, rows0_v)

            @pl.when(c0 + 2 < nchunks)
            def _():
                start_gather(c0 + 2, rows0_v, sem0)

        @pl.when(c1 < nchunks)
        def _():
            wait_gather(rows1_v, sem1)
            proc_chunk(c1, rows1_v)

            @pl.when(c1 + 2 < nchunks)
            def _():
                start_gather(c1 + 2, rows1_v, sem1)

        return 0

    lax.fori_loop(0, (nchunks + 1) // 2, cbody, 0)

    OCK = 64

    def out_chunk(oc, _):
        def row_body(r, _):
            for j in range(D // 16):
                v = agg_v[pl.ds((oc * OCK + r) * D + j * 16, 16)]
                rows0_v[r, pl.ds(j * 16, 16)] = jnp.where(
                    v == -jnp.inf, 0.0, v)
            return 0

        lax.fori_loop(0, OCK, row_body, 0)
        pltpu.sync_copy(
            rows0_v.at[pl.ds(0, OCK)],
            agg_hbm.at[pl.ds(wid * NPT + oc * OCK, OCK)])
        return 0

    lax.fori_loop(0, NPT // OCK, out_chunk, 0)


BR = 1024


def _mm_r_body(h_ref, wrt_ref, bl_ref, o_ref):
    o_ref[...] = jnp.dot(
        h_ref[...], wrt_ref[...], preferred_element_type=jnp.float32,
        precision="highest") + bl_ref[...]


def _mm_r(h, wrt, bl):
    return pl.pallas_call(
        _mm_r_body,
        grid=(NPAD // BR,),
        in_specs=[
            pl.BlockSpec((BR, D), lambda i: (i, 0)),
            pl.BlockSpec((D, D), lambda i: (0, 0)),
            pl.BlockSpec((1, D), lambda i: (0, 0)),
        ],
        out_specs=pl.BlockSpec((BR, D), lambda i: (i, 0)),
        out_shape=jax.ShapeDtypeStruct((NPAD, D), jnp.float32),
    )(h, wrt, bl)


def _mm_f_body(relu, agg_ref, r_ref, wlt_ref, o_ref):
    acc = jnp.dot(agg_ref[...], wlt_ref[...],
                  preferred_element_type=jnp.float32, precision="highest")
    acc += r_ref[...]
    if relu:
        acc = jnp.maximum(acc, 0.0)
    o_ref[...] = acc


def _mm_f(agg, r, wlt, relu):
    return pl.pallas_call(
        functools.partial(_mm_f_body, relu),
        grid=(NPAD // BR,),
        in_specs=[
            pl.BlockSpec((BR, D), lambda i: (i, 0)),
            pl.BlockSpec((BR, D), lambda i: (i, 0)),
            pl.BlockSpec((D, D), lambda i: (0, 0)),
        ],
        out_specs=pl.BlockSpec((BR, D), lambda i: (i, 0)),
        out_shape=jax.ShapeDtypeStruct((NPAD, D), jnp.float32),
    )(agg, r, wlt)


def kernel(x, edge_index, params):
    h = jnp.pad(x, ((0, NPAD - N), (0, 0)))
    src_list, dl_list, counts = _prep(edge_index[0], edge_index[1])
    for i, (Wl, bl, Wr) in enumerate(params):
        agg = _segmax(h, src_list, dl_list, counts)
        r = _mm_r(h, Wr.T, bl.reshape(1, D))
        h = _mm_f(agg, r, Wl.T, i < NUM_LAYERS - 1)
    return h[:N]
